# baseline (speedup 1.0000x reference)
import functools

import jax
import jax.numpy as jnp
from jax.experimental import pallas as pl
from jax.experimental.pallas import tpu as pltpu

_N = 10000
_D = 128
_K = 16
_BLOCK = 2048
_NB = (_N + _BLOCK - 1) // _BLOCK


def _dmon_block_kernel(feat_ref, w_ref, b_ref, assign_ref, pooled_ref,
                       s_acc, cs_acc):
    i = pl.program_id(0)
    base = i * _BLOCK
    row_ok = jax.lax.broadcasted_iota(jnp.int32, (_BLOCK, 1), 0) < _N - base
    feat = jnp.where(row_ok, feat_ref[...], 0.0)

    bias = b_ref[...].T
    logits_t = jax.lax.dot_general(
        w_ref[...], feat, (((1,), (1,)), ((), ())),
        preferred_element_type=jnp.float32) + bias
    e = jnp.exp(logits_t)
    a_t = e / jnp.sum(e, axis=0, keepdims=True)
    col_ok = jax.lax.broadcasted_iota(
        jnp.int32, (1, _BLOCK), 1) < _N - base
    a_t = jnp.where(col_ok, a_t, 0.0)
    assign_ref[...] = a_t

    part = jax.lax.dot_general(
        a_t.astype(jnp.bfloat16), feat.astype(jnp.bfloat16),
        (((1,), (0,)), ((), ())),
        preferred_element_type=jnp.float32)
    cs_part = jnp.sum(a_t, axis=1, keepdims=True)

    @pl.when(i == 0)
    def _():
        s_acc[...] = part
        cs_acc[...] = cs_part

    @pl.when(i > 0)
    def _():
        s_acc[...] = s_acc[...] + part
        cs_acc[...] = cs_acc[...] + cs_part

    @pl.when(i == _NB - 1)
    def _():
        pooled = s_acc[...] / cs_acc[...]
        scale = 1.0507009873554805
        alpha = 1.6732632423543772
        pooled_ref[...] = scale * jnp.where(
            pooled > 0, pooled, alpha * (jnp.exp(pooled) - 1.0))


def kernel(features, edge_index, edge_vals, W, b):
    del edge_index, edge_vals
    b_row = b.reshape(1, _K)
    assignments_t, features_pooled = pl.pallas_call(
        _dmon_block_kernel,
        grid=(_NB,),
        in_specs=[
            pl.BlockSpec((_BLOCK, _D), lambda i: (i, 0)),
            pl.BlockSpec((_K, _D), lambda i: (0, 0)),
            pl.BlockSpec((1, _K), lambda i: (0, 0)),
        ],
        out_specs=[
            pl.BlockSpec((_K, _BLOCK), lambda i: (0, i)),
            pl.BlockSpec((_K, _D), lambda i: (0, 0)),
        ],
        out_shape=[
            jax.ShapeDtypeStruct((_K, _N), jnp.float32),
            jax.ShapeDtypeStruct((_K, _D), jnp.float32),
        ],
        scratch_shapes=[
            pltpu.VMEM((_K, _D), jnp.float32),
            pltpu.VMEM((_K, 1), jnp.float32),
        ],
        compiler_params=pltpu.CompilerParams(
            vmem_limit_bytes=57 * 1024 * 1024),
    )(features, W, b_row)
    return (features_pooled, assignments_t.T)
